# baseline (speedup 1.0000x reference)
import functools

import jax
import jax.numpy as jnp
from jax import lax
from jax.experimental import pallas as pl
from jax.experimental.pallas import tpu as pltpu
from jax.experimental.pallas import tpu_sc as plsc

_NC = 2
_NS = 16
_NW = _NC * _NS
_D = 64
_C = 512


@functools.lru_cache(maxsize=None)
def _build(B):
    assert B % (_NW * _C) == 0
    b_per_w = B // _NW
    n_chunks = b_per_w // _C
    mesh = plsc.VectorSubcoreMesh(
        core_axis_name="c", subcore_axis_name="s",
        num_cores=_NC, num_subcores=_NS)

    @functools.partial(
        pl.kernel,
        out_type=jax.ShapeDtypeStruct((B, _D), jnp.float32),
        mesh=mesh,
        scratch_types=[
            pltpu.VMEM((_C,), jnp.int32),
            pltpu.VMEM((_C, _D), jnp.float32),
            pltpu.SemaphoreType.DMA,
        ],
        compiler_params=pltpu.CompilerParams(use_tc_tiling_on_sc=False),
    )
    def gather_kernel(idx_hbm, table_hbm, out_hbm, idx_v, rows_v, sem):
        wid = lax.axis_index("s") * _NC + lax.axis_index("c")
        base = wid * b_per_w

        def body(i, _):
            off = pl.multiple_of(base + i * _C, _C)
            pltpu.sync_copy(idx_hbm.at[pl.ds(off, _C)], idx_v)
            pltpu.async_copy(table_hbm.at[idx_v], rows_v, sem).wait()
            pltpu.sync_copy(rows_v, out_hbm.at[pl.ds(off, _C)])
            return ()

        lax.fori_loop(0, n_chunks, body, (), unroll=False)

    return gather_kernel


@jax.jit
def kernel(inputs, embedding):
    B = inputs.size
    flat = inputs.reshape(B)
    out = _build(B)(flat, embedding)
    return out.reshape(inputs.shape + (embedding.shape[1],))


# speedup vs baseline: 1.0432x; 1.0432x over previous
import functools

import jax
import jax.numpy as jnp
from jax import lax
from jax.experimental import pallas as pl
from jax.experimental.pallas import tpu as pltpu
from jax.experimental.pallas import tpu_sc as plsc

_NC = 2
_NS = 16
_NW = _NC * _NS
_D = 64
_C = 512


@functools.lru_cache(maxsize=None)
def _build(B):
    assert B % (_NW * 2 * _C) == 0
    b_per_w = B // _NW
    n_chunks = b_per_w // _C
    mesh = plsc.VectorSubcoreMesh(
        core_axis_name="c", subcore_axis_name="s",
        num_cores=_NC, num_subcores=_NS)

    @functools.partial(
        pl.kernel,
        out_type=jax.ShapeDtypeStruct((B, _D), jnp.float32),
        mesh=mesh,
        scratch_types=[
            pltpu.VMEM((b_per_w,), jnp.int32),
            pltpu.VMEM((_C, _D), jnp.float32),
            pltpu.VMEM((_C, _D), jnp.float32),
            pltpu.SemaphoreType.DMA,
            pltpu.SemaphoreType.DMA,
            pltpu.SemaphoreType.DMA,
            pltpu.SemaphoreType.DMA,
        ],
        compiler_params=pltpu.CompilerParams(use_tc_tiling_on_sc=False),
    )
    def gather_kernel(idx_hbm, table_hbm, out_hbm,
                      idx_v, rows0, rows1, sg0, sg1, so0, so1):
        wid = lax.axis_index("s") * _NC + lax.axis_index("c")
        base = wid * b_per_w
        bufs = ((rows0, sg0, so0), (rows1, sg1, so1))

        pltpu.sync_copy(idx_hbm.at[pl.ds(pl.multiple_of(base, _C), b_per_w)],
                        idx_v)

        def gather(g, b):
            rows, sg, _ = bufs[b]
            loc = pl.multiple_of(g * _C, _C)
            return pltpu.make_async_copy(
                table_hbm.at[idx_v.at[pl.ds(loc, _C)]], rows, sg)

        def writeback(g, b):
            rows, _, so = bufs[b]
            off = pl.multiple_of(base + g * _C, _C)
            return pltpu.make_async_copy(rows, out_hbm.at[pl.ds(off, _C)], so)

        def outer(k, _):
            for j in range(2):
                g = 2 * k + j
                b = j
                bp = 1 - j

                @pl.when(k >= 1)
                def _():
                    writeback(g - 2, b).wait()
                gather(g, b).start()

                if j == 0:
                    @pl.when(k >= 1)
                    def _():
                        gather(g - 1, bp).wait()
                        writeback(g - 1, bp).start()
                else:
                    gather(g - 1, bp).wait()
                    writeback(g - 1, bp).start()
            return ()

        lax.fori_loop(0, n_chunks // 2, outer, (), unroll=False)

        gather(n_chunks - 1, 1).wait()
        writeback(n_chunks - 1, 1).start()
        writeback(n_chunks - 2, 0).wait()
        writeback(n_chunks - 1, 1).wait()

    return gather_kernel


@jax.jit
def kernel(inputs, embedding):
    B = inputs.size
    flat = inputs.reshape(B)
    out = _build(B)(flat, embedding)
    return out.reshape(inputs.shape + (embedding.shape[1],))
